# initial kernel scaffold (unmeasured)
import jax
import jax.numpy as jnp
from jax import lax
from jax.experimental import pallas as pl
from jax.experimental.pallas import tpu as pltpu


def kernel(
    x,
):
    def body(*refs):
        pass

    out_shape = jax.ShapeDtypeStruct(..., jnp.float32)
    return pl.pallas_call(body, out_shape=out_shape)(...)


# baseline (device time: 26813 ns/iter reference)
import jax
import jax.numpy as jnp
from jax import lax
from jax.experimental import pallas as pl
from jax.experimental.pallas import tpu as pltpu

N_DEV = 32


def kernel(x):
    m_per, n_per = x.shape

    def body(x_ref, o_ref, stats_ref, send_sems, recv_sems):
        me = lax.axis_index("i")

        xv = x_ref[:, :]
        m_loc = jnp.max(xv, axis=1)
        e = jnp.exp(xv - m_loc[:, None])
        s_loc = jnp.sum(e, axis=1)
        stats_ref[me, pl.ds(0, m_per)] = m_loc
        stats_ref[me, pl.ds(m_per, m_per)] = s_loc

        o_ref[:, :] = e

        barrier_sem = pltpu.get_barrier_semaphore()
        for k in range(1, N_DEV):
            pl.semaphore_signal(
                barrier_sem, inc=1,
                device_id=((me + k) % N_DEV,),
                device_id_type=pl.DeviceIdType.MESH,
            )
        pl.semaphore_wait(barrier_sem, N_DEV - 1)

        rdmas = []
        for k in range(1, N_DEV):
            rdma = pltpu.make_async_remote_copy(
                src_ref=stats_ref.at[me],
                dst_ref=stats_ref.at[me],
                send_sem=send_sems.at[k - 1],
                recv_sem=recv_sems.at[k - 1],
                device_id=((me + k) % N_DEV,),
                device_id_type=pl.DeviceIdType.MESH,
            )
            rdma.start()
            rdmas.append(rdma)

        for rdma in rdmas:
            rdma.wait_recv()

        all_m = stats_ref[:, pl.ds(0, m_per)]
        all_s = stats_ref[:, pl.ds(m_per, m_per)]
        m_g = jnp.max(all_m, axis=0)
        s_g = jnp.sum(all_s * jnp.exp(all_m - m_g[None, :]), axis=0)
        scale = jnp.exp(m_loc - m_g) / s_g
        o_ref[:, :] = o_ref[:, :] * scale[:, None]

        for rdma in rdmas:
            rdma.wait_send()

    return pl.pallas_call(
        body,
        out_shape=jax.ShapeDtypeStruct((m_per, n_per), jnp.float32),
        in_specs=[pl.BlockSpec(memory_space=pltpu.VMEM)],
        out_specs=pl.BlockSpec(memory_space=pltpu.VMEM),
        scratch_shapes=[
            pltpu.VMEM((N_DEV, 2 * m_per), jnp.float32),
            pltpu.SemaphoreType.DMA((N_DEV - 1,)),
            pltpu.SemaphoreType.DMA((N_DEV - 1,)),
        ],
        compiler_params=pltpu.CompilerParams(collective_id=0),
    )(x)


# device time: 23806 ns/iter; 1.1263x vs baseline; 1.1263x over previous
import jax
import jax.numpy as jnp
from jax import lax
from jax.experimental import pallas as pl
from jax.experimental.pallas import tpu as pltpu

N_DEV = 32


def kernel(x):
    m_per, n_per = x.shape

    def body(x_ref, o_ref, maxs_ref, sums_ref,
             send_m, recv_m, send_s, recv_s):
        me = lax.axis_index("i")

        barrier_sem = pltpu.get_barrier_semaphore()
        for k in range(1, N_DEV):
            pl.semaphore_signal(
                barrier_sem, inc=1,
                device_id=((me + k) % N_DEV,),
                device_id_type=pl.DeviceIdType.MESH,
            )

        xv = x_ref[:, :]
        m_loc = jnp.max(xv, axis=1)
        maxs_ref[me, :] = m_loc

        pl.semaphore_wait(barrier_sem, N_DEV - 1)

        m_rdmas = []
        for k in range(1, N_DEV):
            rdma = pltpu.make_async_remote_copy(
                src_ref=maxs_ref.at[me],
                dst_ref=maxs_ref.at[me],
                send_sem=send_m.at[k - 1],
                recv_sem=recv_m.at[k - 1],
                device_id=((me + k) % N_DEV,),
                device_id_type=pl.DeviceIdType.MESH,
            )
            rdma.start()
            m_rdmas.append(rdma)

        e = jnp.exp(xv - m_loc[:, None])
        s_loc = jnp.sum(e, axis=1)
        sums_ref[me, :] = s_loc
        o_ref[:, :] = e

        s_rdmas = []
        for k in range(1, N_DEV):
            rdma = pltpu.make_async_remote_copy(
                src_ref=sums_ref.at[me],
                dst_ref=sums_ref.at[me],
                send_sem=send_s.at[k - 1],
                recv_sem=recv_s.at[k - 1],
                device_id=((me + k) % N_DEV,),
                device_id_type=pl.DeviceIdType.MESH,
            )
            rdma.start()
            s_rdmas.append(rdma)

        for rdma in m_rdmas:
            rdma.wait_recv()
        all_m = maxs_ref[:, :]
        m_g = jnp.max(all_m, axis=0)
        w = jnp.exp(all_m - m_g[None, :])

        for rdma in s_rdmas:
            rdma.wait_recv()
        s_g = jnp.sum(sums_ref[:, :] * w, axis=0)
        scale = jnp.exp(m_loc - m_g) / s_g
        o_ref[:, :] = o_ref[:, :] * scale[:, None]

        for rdma in m_rdmas:
            rdma.wait_send()
        for rdma in s_rdmas:
            rdma.wait_send()

    return pl.pallas_call(
        body,
        out_shape=jax.ShapeDtypeStruct((m_per, n_per), jnp.float32),
        in_specs=[pl.BlockSpec(memory_space=pltpu.VMEM)],
        out_specs=pl.BlockSpec(memory_space=pltpu.VMEM),
        scratch_shapes=[
            pltpu.VMEM((N_DEV, m_per), jnp.float32),
            pltpu.VMEM((N_DEV, m_per), jnp.float32),
            pltpu.SemaphoreType.DMA((N_DEV - 1,)),
            pltpu.SemaphoreType.DMA((N_DEV - 1,)),
            pltpu.SemaphoreType.DMA((N_DEV - 1,)),
            pltpu.SemaphoreType.DMA((N_DEV - 1,)),
        ],
        compiler_params=pltpu.CompilerParams(collective_id=0),
    )(x)
